# baseline (speedup 1.0000x reference)
import jax
import jax.numpy as jnp
from jax import lax
from jax.experimental import pallas as pl
from jax.experimental.pallas import tpu as pltpu
from jax.experimental.pallas import tpu_sc as plsc

_NUM_NODES = 110000
_NUM_PINS = 3200000
_NW = 32
_PPW = _NUM_PINS // _NW
_C = 4000
_NCH = _PPW // _C
_H = _C // 2
_L = 16
_W = 8192


def _body(nx_hbm, ny_hbm, offx_hbm, offy_hbm, p2n_hbm, out_hbm,
          winx, winy,
          idx_a, offx_a, offy_a, bufx_a, bufy_a,
          idx_b, offx_b, offy_b, bufx_b, bufy_b,
          tiny_v, sem_a, sem_b, sem_oa, sem_ob):
  wid = lax.axis_index("s") * 2 + lax.axis_index("c")
  wbase = wid * _PPW

  pltpu.sync_copy(p2n_hbm.at[pl.ds(wbase, _L)], tiny_v)
  m0s = jnp.min(tiny_v[...])
  pltpu.sync_copy(p2n_hbm.at[pl.ds(wbase + _PPW - _L, _L)], tiny_v)
  m1 = jnp.max(tiny_v[...])
  m0 = pl.multiple_of(jnp.bitwise_and(m0s, jnp.int32(-8)), 8)
  wide = (m1 - m0) >= _W

  pltpu.sync_copy(nx_hbm.at[pl.ds(m0, _W)], winx)
  pltpu.sync_copy(ny_hbm.at[pl.ds(m0, _W)], winy)
  mvec = jnp.full((_L,), m0, jnp.int32)

  def in_start(base, idx_h, offx_h, offy_h, sem_h):
    return [
        pltpu.async_copy(p2n_hbm.at[pl.ds(base, _H)], idx_h, sem_h),
        pltpu.async_copy(offx_hbm.at[pl.ds(base, _H)], offx_h, sem_h),
        pltpu.async_copy(offy_hbm.at[pl.ds(base, _H)], offy_h, sem_h),
    ]

  def gather_half(idx_h, offx_h, offy_h, bufx_h, bufy_h):
    for j in range(0, _H, _L):
      s = pl.ds(j, _L)
      li = idx_h[s] - mvec
      bufx_h[s] = plsc.load_gather(winx, [li]) + offx_h[s]
      bufy_h[s] = plsc.load_gather(winy, [li]) + offy_h[s]

  def out_start(base, bufx_h, bufy_h, sem_h):
    return [
        pltpu.async_copy(bufx_h, out_hbm.at[pl.ds(base, _H)], sem_h),
        pltpu.async_copy(bufy_h, out_hbm.at[pl.ds(_NUM_PINS + base, _H)], sem_h),
    ]

  @pl.when(jnp.logical_not(wide))
  def _fast_loop():
    def chunk(i, carry):
      base = pl.multiple_of(wbase + i * _C, 8)
      in_a = in_start(base, idx_a, offx_a, offy_a, sem_a)
      in_b = in_start(base + _H, idx_b, offx_b, offy_b, sem_b)
      for d in in_a:
        d.wait()
      gather_half(idx_a, offx_a, offy_a, bufx_a, bufy_a)
      out_a = out_start(base, bufx_a, bufy_a, sem_oa)
      for d in in_b:
        d.wait()
      gather_half(idx_b, offx_b, offy_b, bufx_b, bufy_b)
      out_b = out_start(base + _H, bufx_b, bufy_b, sem_ob)
      for d in out_a + out_b:
        d.wait()
      return carry

    lax.fori_loop(0, _NCH, chunk, None)

  @pl.when(wide)
  def _slow_loop():
    def chunk(i, carry):
      base = wbase + i * _C
      for h, (idx_h, offx_h, offy_h, bufx_h, bufy_h) in enumerate(
          ((idx_a, offx_a, offy_a, bufx_a, bufy_a),
           (idx_b, offx_b, offy_b, bufx_b, bufy_b))):
        hb = pl.multiple_of(base + h * _H, 8)
        pltpu.sync_copy(p2n_hbm.at[pl.ds(hb, _H)], idx_h)
        pltpu.sync_copy(offx_hbm.at[pl.ds(hb, _H)], offx_h)
        pltpu.sync_copy(offy_hbm.at[pl.ds(hb, _H)], offy_h)
        pltpu.async_copy(nx_hbm.at[idx_h], bufx_h, sem_a).wait()
        pltpu.async_copy(ny_hbm.at[idx_h], bufy_h, sem_b).wait()

        @plsc.parallel_loop(0, _H, _L, unroll=8)
        def _a(j):
          s = pl.ds(j, _L)
          bufx_h[s] = bufx_h[s] + offx_h[s]
          bufy_h[s] = bufy_h[s] + offy_h[s]

        pltpu.sync_copy(bufx_h, out_hbm.at[pl.ds(hb, _H)])
        pltpu.sync_copy(bufy_h, out_hbm.at[pl.ds(_NUM_PINS + hb, _H)])
      return carry

    lax.fori_loop(0, _NCH, chunk, None)


@jax.jit
def kernel(pos, pin_offset_x, pin_offset_y, pin2node_map,
           flat_node2pin_map, flat_node2pin_start_map):
  del flat_node2pin_map, flat_node2pin_start_map
  node_x = pos[:_NUM_NODES]
  node_y = pos[_NUM_NODES:]
  mesh = plsc.VectorSubcoreMesh(core_axis_name="c", subcore_axis_name="s")
  run = pl.kernel(
      _body,
      out_type=jax.ShapeDtypeStruct((2 * _NUM_PINS,), jnp.float32),
      mesh=mesh,
      compiler_params=pltpu.CompilerParams(needs_layout_passes=False),
      scratch_types=[
          pltpu.VMEM((_W,), jnp.float32),
          pltpu.VMEM((_W,), jnp.float32),
          pltpu.VMEM((_H,), jnp.int32),
          pltpu.VMEM((_H,), jnp.float32),
          pltpu.VMEM((_H,), jnp.float32),
          pltpu.VMEM((_H,), jnp.float32),
          pltpu.VMEM((_H,), jnp.float32),
          pltpu.VMEM((_H,), jnp.int32),
          pltpu.VMEM((_H,), jnp.float32),
          pltpu.VMEM((_H,), jnp.float32),
          pltpu.VMEM((_H,), jnp.float32),
          pltpu.VMEM((_H,), jnp.float32),
          pltpu.VMEM((_L,), jnp.int32),
          pltpu.SemaphoreType.DMA,
          pltpu.SemaphoreType.DMA,
          pltpu.SemaphoreType.DMA,
          pltpu.SemaphoreType.DMA,
      ],
  )
  return run(node_x, node_y, pin_offset_x, pin_offset_y, pin2node_map)


# speedup vs baseline: 2.2490x; 2.2490x over previous
import jax
import jax.numpy as jnp
from jax import lax
from jax.experimental import pallas as pl
from jax.experimental.pallas import tpu as pltpu
from jax.experimental.pallas import tpu_sc as plsc

_NUM_NODES = 110000
_NUM_PINS = 3200000
_NW = 32
_PPW = _NUM_PINS // _NW
_C = 20000
_NCH = _PPW // _C
_L = 16
_W = 8192


def _body(nx_hbm, ny_hbm, offx_hbm, offy_hbm, p2n_hbm, out_hbm,
          winx, winy, idx_v, offx_v, offy_v, bufx, bufy,
          tiny_v, semx, semy, semz):
  wid = lax.axis_index("s") * 2 + lax.axis_index("c")
  wbase = wid * _PPW

  pltpu.sync_copy(p2n_hbm.at[pl.ds(wbase, _L)], tiny_v)
  m0s = jnp.min(tiny_v[...])
  pltpu.sync_copy(p2n_hbm.at[pl.ds(wbase + _PPW - _L, _L)], tiny_v)
  m1 = jnp.max(tiny_v[...])
  m0 = pl.multiple_of(jnp.bitwise_and(m0s, jnp.int32(-8)), 8)
  wide = (m1 - m0) >= _W

  wx = pltpu.async_copy(nx_hbm.at[pl.ds(m0, _W)], winx, semx)
  wy = pltpu.async_copy(ny_hbm.at[pl.ds(m0, _W)], winy, semy)
  wx.wait()
  wy.wait()
  mvec = jnp.full((_L,), m0, jnp.int32)

  def chunk(i, carry):
    base = wbase + i * _C
    i0 = pltpu.async_copy(p2n_hbm.at[pl.ds(base, _C)], idx_v, semx)
    i1 = pltpu.async_copy(offx_hbm.at[pl.ds(base, _C)], offx_v, semy)
    i2 = pltpu.async_copy(offy_hbm.at[pl.ds(base, _C)], offy_v, semz)
    i0.wait()
    i1.wait()
    i2.wait()

    @pl.when(jnp.logical_not(wide))
    def _fast():
      @plsc.parallel_loop(0, _C, _L, unroll=8)
      def _g(j):
        s = pl.ds(j, _L)
        li = idx_v[s] - mvec
        bufx[s] = plsc.load_gather(winx, [li]) + offx_v[s]
        bufy[s] = plsc.load_gather(winy, [li]) + offy_v[s]

    @pl.when(wide)
    def _slow():
      pltpu.async_copy(nx_hbm.at[idx_v], bufx, semx).wait()
      pltpu.async_copy(ny_hbm.at[idx_v], bufy, semy).wait()

      @plsc.parallel_loop(0, _C, _L, unroll=8)
      def _a(j):
        s = pl.ds(j, _L)
        bufx[s] = bufx[s] + offx_v[s]
        bufy[s] = bufy[s] + offy_v[s]

    ox = pltpu.async_copy(bufx, out_hbm.at[pl.ds(base, _C)], semx)
    oy = pltpu.async_copy(bufy, out_hbm.at[pl.ds(_NUM_PINS + base, _C)], semy)
    ox.wait()
    oy.wait()
    return carry

  lax.fori_loop(0, _NCH, chunk, None)


@jax.jit
def kernel(pos, pin_offset_x, pin_offset_y, pin2node_map,
           flat_node2pin_map, flat_node2pin_start_map):
  del flat_node2pin_map, flat_node2pin_start_map
  node_x = pos[:_NUM_NODES]
  node_y = pos[_NUM_NODES:]
  mesh = plsc.VectorSubcoreMesh(core_axis_name="c", subcore_axis_name="s")
  run = pl.kernel(
      _body,
      out_type=jax.ShapeDtypeStruct((2 * _NUM_PINS,), jnp.float32),
      mesh=mesh,
      compiler_params=pltpu.CompilerParams(needs_layout_passes=False),
      scratch_types=[
          pltpu.VMEM((_W,), jnp.float32),
          pltpu.VMEM((_W,), jnp.float32),
          pltpu.VMEM((_C,), jnp.int32),
          pltpu.VMEM((_C,), jnp.float32),
          pltpu.VMEM((_C,), jnp.float32),
          pltpu.VMEM((_C,), jnp.float32),
          pltpu.VMEM((_C,), jnp.float32),
          pltpu.VMEM((_L,), jnp.int32),
          pltpu.SemaphoreType.DMA,
          pltpu.SemaphoreType.DMA,
          pltpu.SemaphoreType.DMA,
      ],
  )
  return run(node_x, node_y, pin_offset_x, pin_offset_y, pin2node_map)
